# Initial kernel scaffold; baseline (speedup 1.0000x reference)
import jax
import jax.numpy as jnp
from jax.experimental import pallas as pl


def kernel(x, weight, bias, forget_gate):
    raise NotImplementedError("write your pallas kernel here")


# speedup vs baseline: 24.3763x; 24.3763x over previous
import jax
import jax.numpy as jnp
from jax.experimental import pallas as pl
from jax.experimental.pallas import tpu as pltpu

_EPS = 1e-4


def _ltn_kernel(x_ref, wt_ref, b_ref, f_ref, y_ref, hid_ref, mu_ref, var_ref):
    s = pl.program_id(1)
    sblk, bblk, hid = y_ref.shape

    @pl.when(s == 0)
    def _():
        mu_ref[...] = jnp.zeros_like(mu_ref)
        var_ref[...] = jnp.ones_like(var_ref)

    x2 = x_ref[...].reshape(sblk * bblk, hid)
    y2 = jnp.dot(x2, wt_ref[...], preferred_element_type=jnp.float32) + b_ref[...]
    y_ref[...] = y2.reshape(sblk, bblk, hid)

    f = f_ref[...]

    def body(t, carry):
        mu, var = carry
        y_t = y_ref[t]
        diff = y_t - mu
        inv = jax.lax.rsqrt(var + _EPS)
        y_ref[t] = jnp.tanh(diff * inv)
        mu = mu + f * diff
        var = var + f * (diff * diff - var)
        return (mu, var)

    mu, var = jax.lax.fori_loop(0, sblk, body, (mu_ref[...], var_ref[...]))
    mu_ref[...] = mu
    var_ref[...] = var

    hid_ref[:, :hid] = mu
    hid_ref[:, hid:] = var


def kernel(x, weight, bias, forget_gate):
    S, B, H = x.shape
    SBLK = 128
    BBLK = 16
    wt = weight.T
    b2 = bias.reshape(1, H)
    f2 = forget_gate.reshape(1, H)
    grid = (B // BBLK, S // SBLK)
    y_out, hid = pl.pallas_call(
        _ltn_kernel,
        grid=grid,
        in_specs=[
            pl.BlockSpec((SBLK, BBLK, H), lambda b, s: (s, b, 0)),
            pl.BlockSpec((H, H), lambda b, s: (0, 0)),
            pl.BlockSpec((1, H), lambda b, s: (0, 0)),
            pl.BlockSpec((1, H), lambda b, s: (0, 0)),
        ],
        out_specs=[
            pl.BlockSpec((SBLK, BBLK, H), lambda b, s: (s, b, 0)),
            pl.BlockSpec((BBLK, 2 * H), lambda b, s: (b, 0)),
        ],
        out_shape=[
            jax.ShapeDtypeStruct((S, B, H), jnp.float32),
            jax.ShapeDtypeStruct((B, 2 * H), jnp.float32),
        ],
        scratch_shapes=[
            pltpu.VMEM((BBLK, H), jnp.float32),
            pltpu.VMEM((BBLK, H), jnp.float32),
        ],
        compiler_params=pltpu.CompilerParams(
            dimension_semantics=("parallel", "arbitrary"),
        ),
        name="linear_temp_norm",
    )(x, wt, b2, f2)
    return y_out, hid


# speedup vs baseline: 33.7722x; 1.3854x over previous
import jax
import jax.numpy as jnp
from jax.experimental import pallas as pl
from jax.experimental.pallas import tpu as pltpu

_EPS = 1e-4


def _ltn_kernel(x_ref, wt_ref, b_ref, f_ref, y_ref, hid_ref, mu_ref, var_ref):
    s = pl.program_id(1)
    sblk, bblk, hid = y_ref.shape

    @pl.when(s == 0)
    def _():
        mu_ref[...] = jnp.zeros_like(mu_ref)
        var_ref[...] = jnp.ones_like(var_ref)

    x2 = x_ref[...].reshape(sblk * bblk, hid)
    y2 = jnp.dot(x2, wt_ref[...], preferred_element_type=jnp.float32) + b_ref[...]
    y_ref[...] = y2.reshape(sblk, bblk, hid)

    f = f_ref[...]

    def body(t, carry):
        mu, var = carry
        y_t = y_ref[t]
        diff = y_t - mu
        inv = jax.lax.rsqrt(var + _EPS)
        y_ref[t] = jnp.tanh(diff * inv)
        mu = mu + f * diff
        var = var + f * (diff * diff - var)
        return (mu, var)

    mu, var = jax.lax.fori_loop(0, sblk, body, (mu_ref[...], var_ref[...]),
                                unroll=8)
    mu_ref[...] = mu
    var_ref[...] = var

    hid_ref[:, :hid] = mu
    hid_ref[:, hid:] = var


def kernel(x, weight, bias, forget_gate):
    S, B, H = x.shape
    SBLK = 128
    BBLK = 16
    wt = weight.T
    b2 = bias.reshape(1, H)
    f2 = forget_gate.reshape(1, H)
    grid = (B // BBLK, S // SBLK)
    y_out, hid = pl.pallas_call(
        _ltn_kernel,
        grid=grid,
        in_specs=[
            pl.BlockSpec((SBLK, BBLK, H), lambda b, s: (s, b, 0)),
            pl.BlockSpec((H, H), lambda b, s: (0, 0)),
            pl.BlockSpec((1, H), lambda b, s: (0, 0)),
            pl.BlockSpec((1, H), lambda b, s: (0, 0)),
        ],
        out_specs=[
            pl.BlockSpec((SBLK, BBLK, H), lambda b, s: (s, b, 0)),
            pl.BlockSpec((BBLK, 2 * H), lambda b, s: (b, 0)),
        ],
        out_shape=[
            jax.ShapeDtypeStruct((S, B, H), jnp.float32),
            jax.ShapeDtypeStruct((B, 2 * H), jnp.float32),
        ],
        scratch_shapes=[
            pltpu.VMEM((BBLK, H), jnp.float32),
            pltpu.VMEM((BBLK, H), jnp.float32),
        ],
        compiler_params=pltpu.CompilerParams(
            dimension_semantics=("parallel", "arbitrary"),
        ),
        name="linear_temp_norm",
    )(x, wt, b2, f2)
    return y_out, hid


# speedup vs baseline: 34.4383x; 1.0197x over previous
import jax
import jax.numpy as jnp
from jax.experimental import pallas as pl
from jax.experimental.pallas import tpu as pltpu

_EPS = 1e-4


def _ltn_kernel(x_ref, wt_ref, b_ref, f_ref, y_ref, hid_ref, mu_ref, var_ref):
    s = pl.program_id(1)
    sblk, bblk, hid = y_ref.shape

    @pl.when(s == 0)
    def _():
        mu_ref[...] = jnp.zeros_like(mu_ref)
        var_ref[...] = jnp.ones_like(var_ref)

    x2 = x_ref[...].reshape(sblk * bblk, hid)
    y2 = jnp.dot(x2, wt_ref[...], preferred_element_type=jnp.float32) + b_ref[...]
    y_ref[...] = y2.reshape(sblk, bblk, hid)

    fb = jnp.broadcast_to(f_ref[...], (bblk, hid))

    def body(t, carry):
        mu, var, f = carry
        y_t = y_ref[t]
        diff = y_t - mu
        inv = jax.lax.rsqrt(var + _EPS)
        y_ref[t] = jnp.tanh(diff * inv)
        mu = mu + f * diff
        var = var + f * (diff * diff - var)
        return (mu, var, f)

    mu, var, _ = jax.lax.fori_loop(0, sblk, body,
                                   (mu_ref[...], var_ref[...], fb),
                                   unroll=16)
    mu_ref[...] = mu
    var_ref[...] = var

    hid_ref[:, :hid] = mu
    hid_ref[:, hid:] = var


def kernel(x, weight, bias, forget_gate):
    S, B, H = x.shape
    SBLK = 128
    BBLK = 16
    wt = weight.T
    b2 = bias.reshape(1, H)
    f2 = forget_gate.reshape(1, H)
    grid = (B // BBLK, S // SBLK)
    y_out, hid = pl.pallas_call(
        _ltn_kernel,
        grid=grid,
        in_specs=[
            pl.BlockSpec((SBLK, BBLK, H), lambda b, s: (s, b, 0)),
            pl.BlockSpec((H, H), lambda b, s: (0, 0)),
            pl.BlockSpec((1, H), lambda b, s: (0, 0)),
            pl.BlockSpec((1, H), lambda b, s: (0, 0)),
        ],
        out_specs=[
            pl.BlockSpec((SBLK, BBLK, H), lambda b, s: (s, b, 0)),
            pl.BlockSpec((BBLK, 2 * H), lambda b, s: (b, 0)),
        ],
        out_shape=[
            jax.ShapeDtypeStruct((S, B, H), jnp.float32),
            jax.ShapeDtypeStruct((B, 2 * H), jnp.float32),
        ],
        scratch_shapes=[
            pltpu.VMEM((BBLK, H), jnp.float32),
            pltpu.VMEM((BBLK, H), jnp.float32),
        ],
        compiler_params=pltpu.CompilerParams(
            dimension_semantics=("parallel", "arbitrary"),
        ),
        name="linear_temp_norm",
    )(x, wt, b2, f2)
    return y_out, hid


# speedup vs baseline: 38.1579x; 1.1080x over previous
import jax
import jax.numpy as jnp
from jax.experimental import pallas as pl
from jax.experimental.pallas import tpu as pltpu

_EPS = 1e-4
_GT = 16


def _ltn_kernel(x_ref, wt_ref, b_ref, f_ref, y_ref, hid_ref,
                yscr, mu_ref, var_ref):
    s = pl.program_id(0)
    sblk, batch, hid = y_ref.shape
    groups = sblk // _GT
    rows = _GT * batch

    @pl.when(s == 0)
    def _():
        mu_ref[...] = jnp.zeros_like(mu_ref)
        var_ref[...] = jnp.ones_like(var_ref)

    wt = wt_ref[...]
    bias = b_ref[...]

    yscr[pl.ds(0, rows), :] = (
        jnp.dot(x_ref[pl.ds(0, rows), :], wt,
                preferred_element_type=jnp.float32) + bias)

    fb = jnp.broadcast_to(f_ref[...], (batch, hid))

    def body(g, carry):
        mu, var, f = carry
        for j in range(_GT):
            t = g * _GT + j
            y_t = yscr[pl.ds(t * batch, batch), :]
            diff = y_t - mu
            inv = jax.lax.rsqrt(var + _EPS)
            y_ref[t] = jnp.tanh(diff * inv)
            mu = mu + f * diff
            var = var + f * (diff * diff - var)
        dst = (g + 1) * rows
        src = jnp.minimum(g + 1, groups - 1) * rows
        yscr[pl.ds(dst, rows), :] = (
            jnp.dot(x_ref[pl.ds(src, rows), :], wt,
                    preferred_element_type=jnp.float32) + bias)
        return (mu, var, f)

    mu, var, _ = jax.lax.fori_loop(
        0, groups, body, (mu_ref[...], var_ref[...], fb))
    mu_ref[...] = mu
    var_ref[...] = var

    hid_ref[:, :hid] = mu
    hid_ref[:, hid:] = var


def kernel(x, weight, bias, forget_gate):
    S, B, H = x.shape
    SBLK = 128
    x2 = x.reshape(S * B, H)
    wt = weight.T
    b2 = bias.reshape(1, H)
    f2 = forget_gate.reshape(1, H)
    rows_blk = SBLK * B
    y_out, hid = pl.pallas_call(
        _ltn_kernel,
        grid=(S // SBLK,),
        in_specs=[
            pl.BlockSpec((rows_blk, H), lambda s: (s, 0)),
            pl.BlockSpec((H, H), lambda s: (0, 0)),
            pl.BlockSpec((1, H), lambda s: (0, 0)),
            pl.BlockSpec((1, H), lambda s: (0, 0)),
        ],
        out_specs=[
            pl.BlockSpec((SBLK, B, H), lambda s: (s, 0, 0)),
            pl.BlockSpec((B, 2 * H), lambda s: (0, 0)),
        ],
        out_shape=[
            jax.ShapeDtypeStruct((S, B, H), jnp.float32),
            jax.ShapeDtypeStruct((B, 2 * H), jnp.float32),
        ],
        scratch_shapes=[
            pltpu.VMEM((rows_blk + _GT * B, H), jnp.float32),
            pltpu.VMEM((B, H), jnp.float32),
            pltpu.VMEM((B, H), jnp.float32),
        ],
        compiler_params=pltpu.CompilerParams(
            dimension_semantics=("arbitrary",),
            vmem_limit_bytes=52 * 1024 * 1024,
        ),
        name="linear_temp_norm",
    )(x2, wt, b2, f2)
    return y_out, hid
